# baseline (speedup 1.0000x reference)
import functools

import jax
import jax.numpy as jnp
from jax import lax
from jax.experimental import pallas as pl
from jax.experimental.pallas import tpu as pltpu
from jax.experimental.pallas import tpu_sc as plsc

_N = 10000
_E = 320000
_D = 128
_DE = 16

_H = 1
_EH = _E // _H

_BE = 12800
_NBE = _EH // _BE


def _edge_gate_body(eat_ref, w2_ref, b_ref, s_ref):
    g = jax.lax.dot_general(eat_ref[...], w2_ref[...],
                            dimension_numbers=(((0,), (0,)), ((), ())),
                            preferred_element_type=jnp.float32)
    g = g + b_ref[...]
    s_ref[...] = g * jax.nn.sigmoid(g)


def _edge_gate(ea_t, W2, b2, half):
    return pl.pallas_call(
        _edge_gate_body,
        grid=(_NBE,),
        in_specs=[
            pl.BlockSpec((_DE, _BE), lambda j: (0, j + half * _NBE)),
            pl.BlockSpec((_DE, _D), lambda j: (0, 0)),
            pl.BlockSpec((1, _D), lambda j: (0, 0)),
        ],
        out_specs=pl.BlockSpec((_BE, _D), lambda j: (j, 0)),
        out_shape=jax.ShapeDtypeStruct((_EH, _D), jnp.float32),
    )(ea_t, W2, b2)


_NC = 2
_NS = 16
_CH = 64
_CHH = _EH // _CH
_NCHT = _CHH // (_NC * _NS)
_XTRA = _CHH - _NCHT * _NC * _NS
_RPAD = 10112
_RPT = _RPAD // _NS


_NB = 4
_NGRP = _NCHT // _NB
_NMAIN = _NGRP * _NB
_NPOST = _NCHT - _NMAIN


def _segsum_body(s_hbm, ei_hbm, out_hbm,
                 idx_v, rows_v, lsems, ssems, acc_sh):
    c = lax.axis_index("c")
    s = lax.axis_index("s")
    wid = c * _NS + s
    row0 = s * _RPT
    z16 = jnp.zeros((16,), jnp.float32)

    def zstep(r, carry):
        for k in range(_D // 16):
            rows_v[0, r, pl.ds(k * 16, 16)] = z16
        return carry

    lax.fori_loop(0, _CH, zstep, 0)
    for k in range(_RPT // _CH):
        pltpu.sync_copy(rows_v.at[0], acc_sh.at[pl.ds(row0 + k * _CH, _CH), :])
    _REM = _RPT - (_RPT // _CH) * _CH
    if _REM:
        pltpu.sync_copy(rows_v.at[0, pl.ds(0, _REM), :],
                        acc_sh.at[pl.ds(row0 + (_RPT // _CH) * _CH, _REM), :])
    plsc.subcore_barrier()

    cbase = wid * _NCHT

    def start(q, b):
        pltpu.async_copy(ei_hbm.at[0, pl.ds(pl.multiple_of(q * _CH, _CH), _CH)],
                         idx_v.at[b], lsems.at[b])
        pltpu.async_copy(s_hbm.at[pl.ds(pl.multiple_of(q * _CH, _CH), _CH), :],
                         rows_v.at[b], lsems.at[b])

    def finish(b):
        pltpu.make_async_copy(ei_hbm.at[0, pl.ds(0, _CH)], idx_v.at[b], lsems.at[b]).wait()
        pltpu.make_async_copy(s_hbm.at[pl.ds(0, _CH), :], rows_v.at[b], lsems.at[b]).wait()

    def scatter_async(b):
        pltpu.async_copy(rows_v.at[b], acc_sh.at[idx_v.at[b]], ssems.at[b], add=True)

    def scatter_wait(b):
        pltpu.make_async_copy(rows_v.at[b], acc_sh.at[idx_v.at[b]],
                              ssems.at[b]).wait()

    start(cbase, 0)
    start(cbase + 1, 1)

    def ring(g, carry):
        for u in range(_NB):
            b = u
            b2 = (u + 2) % _NB
            finish(b)
            scatter_async(b)
            if u >= 2:
                scatter_wait(b2)
            else:
                @pl.when(g >= 1)
                def _():
                    scatter_wait(b2)

            @pl.when(g * _NB + u + 2 <= _NCHT - 1)
            def _():
                start(cbase + g * _NB + u + 2, b2)
        return carry

    lax.fori_loop(0, _NGRP, ring, 0)
    scatter_wait((_NMAIN - 2) % _NB)
    scatter_wait((_NMAIN - 1) % _NB)

    @pl.when(wid < _XTRA)
    def _():
        q = _NCHT * _NC * _NS + wid
        pltpu.sync_copy(ei_hbm.at[0, pl.ds(pl.multiple_of(q * _CH, _CH), _CH)],
                        idx_v.at[0])
        pltpu.sync_copy(s_hbm.at[pl.ds(pl.multiple_of(q * _CH, _CH), _CH), :],
                        rows_v.at[0])
        pltpu.sync_copy(rows_v.at[0], acc_sh.at[idx_v.at[0]], add=True)

    plsc.subcore_barrier()
    pltpu.sync_copy(acc_sh.at[pl.ds(row0, _RPT), :],
                    out_hbm.at[c, pl.ds(row0, _RPT), :])


def _segsum(S, edge_index):
    mesh = plsc.VectorSubcoreMesh(core_axis_name="c", subcore_axis_name="s")
    fn = pl.kernel(
        _segsum_body,
        out_type=jax.ShapeDtypeStruct((_NC, _RPAD, _D), jnp.float32),
        mesh=mesh,
        scratch_types=[
            pltpu.VMEM((_NB, _CH), jnp.int32),
            pltpu.VMEM((_NB, _CH, _D), jnp.float32),
            pltpu.SemaphoreType.DMA((_NB,)),
            pltpu.SemaphoreType.DMA((_NB,)),
            pltpu.VMEM_SHARED((_RPAD, _D), jnp.float32),
        ],
    )
    return fn(S, edge_index)


_BN = 2000


def _stats_body(x_ref, w1_ref, pa_ref, t_ref, sum_ref, sq_ref):
    j = pl.program_id(0)
    t = jnp.dot(x_ref[...], w1_ref[...], preferred_element_type=jnp.float32)
    t = t * (pa_ref[0] + pa_ref[1])
    t_ref[...] = t

    @pl.when(j == 0)
    def _():
        sum_ref[...] = jnp.zeros_like(sum_ref)
        sq_ref[...] = jnp.zeros_like(sq_ref)

    sum_ref[...] += jnp.sum(t, axis=0, keepdims=True)
    sq_ref[...] += jnp.sum(t * t, axis=0, keepdims=True)


def _stats(x, W1, pa):
    pspec = pl.BlockSpec((_NC, _BN, _D), lambda j: (0, j, 0))
    return pl.pallas_call(
        _stats_body,
        grid=(_N // _BN,),
        in_specs=[
            pl.BlockSpec((_BN, _D), lambda j: (j, 0)),
            pl.BlockSpec((_D, _D), lambda j: (0, 0)),
            pspec,
        ],
        out_specs=[
            pl.BlockSpec((_BN, _D), lambda j: (j, 0)),
            pl.BlockSpec((1, _D), lambda j: (0, 0)),
            pl.BlockSpec((1, _D), lambda j: (0, 0)),
        ],
        out_shape=[
            jax.ShapeDtypeStruct((_N, _D), jnp.float32),
            jax.ShapeDtypeStruct((1, _D), jnp.float32),
            jax.ShapeDtypeStruct((1, _D), jnp.float32),
        ],
    )(x, W1, pa)


def _norm_body(t_ref, x_ref, sum_ref, sq_ref, gamma_ref, beta_ref, o_ref):
    mean = sum_ref[...] / _N
    var = sq_ref[...] / _N - mean * mean
    rstd = lax.rsqrt(var + 1e-5)
    o_ref[...] = (t_ref[...] - mean) * rstd * gamma_ref[...] + beta_ref[...] + x_ref[...]


def _norm(t, x, ssum, ssq, gamma, beta):
    return pl.pallas_call(
        _norm_body,
        grid=(_N // _BN,),
        in_specs=[
            pl.BlockSpec((_BN, _D), lambda j: (j, 0)),
            pl.BlockSpec((_BN, _D), lambda j: (j, 0)),
            pl.BlockSpec((1, _D), lambda j: (0, 0)),
            pl.BlockSpec((1, _D), lambda j: (0, 0)),
            pl.BlockSpec((1, _D), lambda j: (0, 0)),
            pl.BlockSpec((1, _D), lambda j: (0, 0)),
        ],
        out_specs=pl.BlockSpec((_BN, _D), lambda j: (j, 0)),
        out_shape=jax.ShapeDtypeStruct((_N, _D), jnp.float32),
    )(t, x, ssum, ssq, gamma.reshape(1, _D), beta.reshape(1, _D))


def kernel(x, edge_index, edge_attr, W1, W2, b, gamma, beta):
    ea_t = edge_attr.T
    S = _edge_gate(ea_t, W2, b.reshape(1, _D), 0)
    pa = _segsum(S, edge_index)
    t, ssum, ssq = _stats(x, W1, pa)
    return _norm(t, x, ssum, ssq, gamma, beta)


# speedup vs baseline: 1.1029x; 1.1029x over previous
import functools

import jax
import jax.numpy as jnp
from jax import lax
from jax.experimental import pallas as pl
from jax.experimental.pallas import tpu as pltpu
from jax.experimental.pallas import tpu_sc as plsc

_N = 10000
_E = 320000
_D = 128
_DE = 16

_H = 1
_EH = _E // _H

_BE = 12800
_NBE = _EH // _BE


def _edge_gate_body(eat_ref, w2_ref, b_ref, s_ref):
    g = jax.lax.dot_general(eat_ref[...], w2_ref[...],
                            dimension_numbers=(((0,), (0,)), ((), ())),
                            preferred_element_type=jnp.float32)
    g = g + b_ref[...]
    s_ref[...] = g * jax.nn.sigmoid(g)


def _edge_gate(ea_t, W2, b2, half):
    return pl.pallas_call(
        _edge_gate_body,
        grid=(_NBE,),
        in_specs=[
            pl.BlockSpec((_DE, _BE), lambda j: (0, j + half * _NBE)),
            pl.BlockSpec((_DE, _D), lambda j: (0, 0)),
            pl.BlockSpec((1, _D), lambda j: (0, 0)),
        ],
        out_specs=pl.BlockSpec((_BE, _D), lambda j: (j, 0)),
        out_shape=jax.ShapeDtypeStruct((_EH, _D), jnp.float32),
    )(ea_t, W2, b2)


_NC = 2
_NS = 16
_CH = 128
_CHH = _EH // _CH
_NCHT = _CHH // (_NC * _NS)
_XTRA = _CHH - _NCHT * _NC * _NS
_RPAD = 10112
_RPT = _RPAD // _NS


_NB = 2
_NGRP = _NCHT // _NB


def _segsum_body(s_hbm, ei_hbm, out_hbm,
                 idx_v, rows_v, lsems, acc_sh):
    c = lax.axis_index("c")
    s = lax.axis_index("s")
    wid = c * _NS + s
    row0 = s * _RPT
    z16 = jnp.zeros((16,), jnp.float32)

    def zstep(r, carry):
        for k in range(_D // 16):
            rows_v[0, r, pl.ds(k * 16, 16)] = z16
        return carry

    lax.fori_loop(0, _CH, zstep, 0)
    for k in range(_RPT // _CH):
        pltpu.sync_copy(rows_v.at[0], acc_sh.at[pl.ds(row0 + k * _CH, _CH), :])
    _REM = _RPT - (_RPT // _CH) * _CH
    if _REM:
        pltpu.sync_copy(rows_v.at[0, pl.ds(0, _REM), :],
                        acc_sh.at[pl.ds(row0 + (_RPT // _CH) * _CH, _REM), :])
    plsc.subcore_barrier()

    cbase = wid * _NCHT

    def start(q, b):
        pltpu.async_copy(ei_hbm.at[0, pl.ds(pl.multiple_of(q * _CH, _CH), _CH)],
                         idx_v.at[b], lsems.at[b])
        pltpu.async_copy(s_hbm.at[pl.ds(pl.multiple_of(q * _CH, _CH), _CH), :],
                         rows_v.at[b], lsems.at[b])

    def finish(b):
        pltpu.make_async_copy(ei_hbm.at[0, pl.ds(0, _CH)], idx_v.at[b], lsems.at[b]).wait()
        pltpu.make_async_copy(s_hbm.at[pl.ds(0, _CH), :], rows_v.at[b], lsems.at[b]).wait()

    def scatter(b):
        pltpu.sync_copy(rows_v.at[b], acc_sh.at[idx_v.at[b]], add=True)

    start(cbase, 0)
    start(cbase + 1, 1)

    def ring(g, carry):
        for b in range(_NB):
            finish(b)
            scatter(b)

            @pl.when(g * _NB + b + 2 <= _NCHT - 1)
            def _():
                start(cbase + g * _NB + b + 2, b)
        return carry

    lax.fori_loop(0, _NGRP, ring, 0)

    @pl.when(wid < _XTRA)
    def _():
        q = _NCHT * _NC * _NS + wid
        pltpu.sync_copy(ei_hbm.at[0, pl.ds(pl.multiple_of(q * _CH, _CH), _CH)],
                        idx_v.at[0])
        pltpu.sync_copy(s_hbm.at[pl.ds(pl.multiple_of(q * _CH, _CH), _CH), :],
                        rows_v.at[0])
        pltpu.sync_copy(rows_v.at[0], acc_sh.at[idx_v.at[0]], add=True)

    plsc.subcore_barrier()
    pltpu.sync_copy(acc_sh.at[pl.ds(row0, _RPT), :],
                    out_hbm.at[c, pl.ds(row0, _RPT), :])


def _segsum(S, edge_index):
    mesh = plsc.VectorSubcoreMesh(core_axis_name="c", subcore_axis_name="s")
    fn = pl.kernel(
        _segsum_body,
        out_type=jax.ShapeDtypeStruct((_NC, _RPAD, _D), jnp.float32),
        mesh=mesh,
        scratch_types=[
            pltpu.VMEM((_NB, _CH), jnp.int32),
            pltpu.VMEM((_NB, _CH, _D), jnp.float32),
            pltpu.SemaphoreType.DMA((_NB,)),
            pltpu.VMEM_SHARED((_RPAD, _D), jnp.float32),
        ],
    )
    return fn(S, edge_index)


def _bn_body(x_ref, w1_ref, p_ref, gamma_ref, beta_ref, o_ref):
    t = jnp.dot(x_ref[...], w1_ref[...], preferred_element_type=jnp.float32)
    t = t * (p_ref[0, :_N, :] + p_ref[1, :_N, :])
    mean = jnp.mean(t, axis=0, keepdims=True)
    var = jnp.mean(t * t, axis=0, keepdims=True) - mean * mean
    rstd = lax.rsqrt(var + 1e-5)
    o_ref[...] = (t - mean) * rstd * gamma_ref[...] + beta_ref[...] + x_ref[...]


def _bn(x, W1, partials, gamma, beta):
    return pl.pallas_call(
        _bn_body,
        in_specs=[
            pl.BlockSpec((_N, _D), lambda: (0, 0)),
            pl.BlockSpec((_D, _D), lambda: (0, 0)),
            pl.BlockSpec((_NC, _RPAD, _D), lambda: (0, 0, 0)),
            pl.BlockSpec((1, _D), lambda: (0, 0)),
            pl.BlockSpec((1, _D), lambda: (0, 0)),
        ],
        out_specs=pl.BlockSpec((_N, _D), lambda: (0, 0)),
        out_shape=jax.ShapeDtypeStruct((_N, _D), jnp.float32),
    )(x, W1, partials, gamma.reshape(1, _D), beta.reshape(1, _D))


def kernel(x, edge_index, edge_attr, W1, W2, b, gamma, beta):
    ea_t = edge_attr.T
    S = _edge_gate(ea_t, W2, b.reshape(1, _D), 0)
    pa = _segsum(S, edge_index)
    return _bn(x, W1, pa, gamma, beta)
